# Initial kernel scaffold; baseline (speedup 1.0000x reference)
import jax
import jax.numpy as jnp
from jax.experimental import pallas as pl


def kernel(x, edge_index, edge_attr, batch_idx, W_in, b_in, W_rel, W_root, b_rgcn, W_l, b_l, W_r, W_h1, b_h1, W_h2, b_h2):
    raise NotImplementedError("write your pallas kernel here")


# speedup vs baseline: 6.2263x; 6.2263x over previous
import jax
import jax.numpy as jnp
from jax import lax
from jax.experimental import pallas as pl
from jax.experimental.pallas import tpu as pltpu
from jax.experimental.pallas import tpu_sc as plsc

N = 10000
E = 320000
D = 128
R = 4
G = 64
MAXDEG = 10

NC, NS = 2, 16
NW = NC * NS
CH = 128
NCH = 79
CHM = 64
NCHM = 158
EPT = NCH * CH
EPAD = NW * EPT
RSP = 10240
HB = R * RSP
PAD_DST = N

_f32 = jnp.float32


def _mesh():
  return plsc.VectorSubcoreMesh(
      core_axis_name="c", subcore_axis_name="s",
      num_cores=NC, num_subcores=NS)


_SC_PARAMS = pltpu.CompilerParams(use_tc_tiling_on_sc=False)


def _fill_rows(ref, nrows, ncol16, value):
  v = jnp.full((16,), value, _f32)

  @pl.loop(0, nrows)
  def _(i):
    for k in range(ncol16):
      ref[i, pl.ds(k * 16, 16)] = v


def _hist_body(etR, dstR, out, et2, dst2, idx2, ones_v, z_v, dmp_v, hist_sh,
               hsem):
  cid = lax.axis_index("c")
  sid = lax.axis_index("s")
  w = cid * NS + sid
  myrow = sid * (HB // NS)
  _fill_rows(ones_v, CH, 1, 1.0)
  _fill_rows(z_v, CH, 1, 0.0)

  @pl.loop(0, HB // NS // CH)
  def _(j):
    pltpu.sync_copy(z_v, hist_sh.at[pl.ds(myrow + j * CH, CH)])

  pltpu.sync_copy(etR.at[w], et2)
  pltpu.sync_copy(dstR.at[w], dst2)

  @pl.loop(0, NCH)
  def _(j):
    for k in range(CH // 16):
      sl = pl.ds(k * 16, 16)
      idx2[j, sl] = et2[j, sl] * RSP + dst2[j, sl]

  plsc.subcore_barrier()

  @pl.loop(0, NCH)
  def _(j):
    pltpu.async_copy(ones_v, hist_sh.at[idx2.at[j]], hsem, add=True)

    @pl.when(j >= 8)
    def _():
      pltpu.make_async_copy(ones_v, hist_sh.at[idx2.at[j - 8]], hsem).wait()

  @pl.loop(NCH - 8, NCH)
  def _(j):
    pltpu.make_async_copy(ones_v, hist_sh.at[idx2.at[j]], hsem).wait()

  plsc.subcore_barrier()

  @pl.loop(0, HB // NS // CH)
  def _(j):
    pltpu.sync_copy(hist_sh.at[pl.ds(myrow + j * CH, CH)], dmp_v)
    pltpu.sync_copy(dmp_v, out.at[cid, pl.ds(myrow + j * CH, CH)])


def _k_hist(mesh):
  return pl.kernel(
      _hist_body,
      out_type=jax.ShapeDtypeStruct((NC, HB, 16), _f32),
      mesh=mesh,
      compiler_params=_SC_PARAMS,
      scratch_types=[
          pltpu.VMEM((NCH, CH), jnp.int32),
          pltpu.VMEM((NCH, CH), jnp.int32),
          pltpu.VMEM((NCH, CH), jnp.int32),
          pltpu.VMEM((CH, 16), _f32),
          pltpu.VMEM((CH, 16), _f32),
          pltpu.VMEM((CH, 16), _f32),
          pltpu.VMEM_SHARED((HB, 16), _f32),
          pltpu.SemaphoreType.DMA,
      ])


def _gs_pipeline(tbl, idxg_row, idxs_row, rows, acc_sh, gsem, ssem, nch):
  pltpu.async_copy(tbl.at[idxg_row(0)], rows.at[0], gsem.at[0])

  @pl.loop(0, nch)
  def _(j):
    b = lax.rem(j, 2)
    nb = 1 - b

    @pl.when(j >= 1)
    def _():
      pltpu.make_async_copy(
          rows.at[nb], acc_sh.at[idxs_row(j - 1)], ssem.at[nb]).wait()

    @pl.when(j + 1 < nch)
    def _():
      pltpu.async_copy(tbl.at[idxg_row(j + 1)], rows.at[nb], gsem.at[nb])

    pltpu.make_async_copy(tbl.at[idxg_row(j)], rows.at[b], gsem.at[b]).wait()
    pltpu.async_copy(rows.at[b], acc_sh.at[idxs_row(j)], ssem.at[b], add=True)

  lb = (nch - 1) % 2
  pltpu.make_async_copy(
      rows.at[lb], acc_sh.at[idxs_row(nch - 1)], ssem.at[lb]).wait()


def _rgcn_body(tbl, etR, srcR, dstR, out, et2, src2, dst2, rows,
               acc_sh, gsem, ssem):
  cid = lax.axis_index("c")
  sid = lax.axis_index("s")
  w = cid * NS + sid
  myrow = sid * (HB // NS)
  pltpu.sync_copy(etR.at[w], et2)
  pltpu.sync_copy(srcR.at[w], src2)
  pltpu.sync_copy(dstR.at[w], dst2)

  @pl.loop(0, NCH)
  def _(j):
    for k in range(CH // 16):
      sl = pl.ds(k * 16, 16)
      et16 = et2[j, sl]
      dst2[j, sl] = et16 * RSP + dst2[j, sl]
      src2[j, sl] = et16 * (4 * N) + src2[j, sl] * 4

  for dc in range(4):
    if dc > 0:
      @pl.loop(0, NCH)
      def _(j):
        for k in range(CH // 16):
          sl = pl.ds(k * 16, 16)
          src2[j, sl] = src2[j, sl] + 1

    _fill_rows(rows.at[0], CH, 2, 0.0)

    @pl.loop(0, HB // NS // CH)
    def _(j):
      pltpu.sync_copy(rows.at[0], acc_sh.at[pl.ds(myrow + j * CH, CH)])

    plsc.subcore_barrier()
    _gs_pipeline(tbl, lambda j: src2.at[j], lambda j: dst2.at[j],
                 rows, acc_sh, gsem, ssem, NCH)
    plsc.subcore_barrier()

    @pl.loop(0, HB // NS // CH)
    def _(j):
      pltpu.sync_copy(acc_sh.at[pl.ds(myrow + j * CH, CH)], rows.at[0])
      pltpu.sync_copy(rows.at[0], out.at[cid, dc, pl.ds(myrow + j * CH, CH)])


def _k_rgcn(mesh):
  return pl.kernel(
      _rgcn_body,
      out_type=jax.ShapeDtypeStruct((NC, 4, HB, 32), _f32),
      mesh=mesh,
      compiler_params=_SC_PARAMS,
      scratch_types=[
          pltpu.VMEM((NCH, CH), jnp.int32),
          pltpu.VMEM((NCH, CH), jnp.int32),
          pltpu.VMEM((NCH, CH), jnp.int32),
          pltpu.VMEM((2, CH, 32), _f32),
          pltpu.VMEM_SHARED((HB, 32), _f32),
          pltpu.SemaphoreType.DMA((2,)),
          pltpu.SemaphoreType.DMA((2,)),
      ])


def _mf_body(tbl, srcR, dstR, out, src2, dst2, rows, acc_sh, gsem, ssem):
  cid = lax.axis_index("c")
  sid = lax.axis_index("s")
  w = cid * NS + sid
  myrow = sid * (RSP // NS)
  pltpu.sync_copy(srcR.at[w], src2)
  pltpu.sync_copy(dstR.at[w], dst2)
  _fill_rows(rows.at[0], CHM, D // 16, 0.0)

  @pl.loop(0, RSP // NS // CHM)
  def _(j):
    pltpu.sync_copy(rows.at[0], acc_sh.at[pl.ds(myrow + j * CHM, CHM)])

  plsc.subcore_barrier()
  _gs_pipeline(tbl, lambda j: src2.at[j], lambda j: dst2.at[j],
               rows, acc_sh, gsem, ssem, NCHM)
  plsc.subcore_barrier()

  @pl.loop(0, RSP // NS // CHM)
  def _(j):
    pltpu.sync_copy(acc_sh.at[pl.ds(myrow + j * CHM, CHM)], rows.at[0])
    pltpu.sync_copy(rows.at[0], out.at[cid, pl.ds(myrow + j * CHM, CHM)])


def _k_mf(mesh):
  return pl.kernel(
      _mf_body,
      out_type=jax.ShapeDtypeStruct((NC, RSP, D), _f32),
      mesh=mesh,
      compiler_params=_SC_PARAMS,
      scratch_types=[
          pltpu.VMEM((NCHM, CHM), jnp.int32),
          pltpu.VMEM((NCHM, CHM), jnp.int32),
          pltpu.VMEM((2, CHM, D), _f32),
          pltpu.VMEM_SHARED((RSP, D), _f32),
          pltpu.SemaphoreType.DMA((2,)),
          pltpu.SemaphoreType.DMA((2,)),
      ])


def _tc_in(x, W, b):
  RB = 1000

  def body(x_ref, w_ref, b_ref, o_ref):
    o_ref[...] = (jnp.dot(x_ref[...], w_ref[...],
                          preferred_element_type=_f32) + b_ref[...])

  return pl.pallas_call(
      body, grid=(N // RB,),
      in_specs=[pl.BlockSpec((RB, D), lambda j: (j, 0)),
                pl.BlockSpec((D, D), lambda j: (0, 0)),
                pl.BlockSpec((1, D), lambda j: (0, 0))],
      out_specs=pl.BlockSpec((RB, D), lambda j: (j, 0)),
      out_shape=jax.ShapeDtypeStruct((N, D), _f32),
  )(x, W, b.reshape(1, D))


def _tc_tables(h, Wrel):
  RB = 1000

  def body(h_ref, w_ref, o_ref):
    a = jnp.maximum(h_ref[...], 0.0)
    o_ref[...] = jnp.dot(a, w_ref[0], preferred_element_type=_f32)

  return pl.pallas_call(
      body, grid=(R, N // RB),
      in_specs=[pl.BlockSpec((RB, D), lambda r, j: (j, 0)),
                pl.BlockSpec((1, D, D), lambda r, j: (r, 0, 0))],
      out_specs=pl.BlockSpec((RB, D), lambda r, j: (r * (N // RB) + j, 0)),
      out_shape=jax.ShapeDtypeStruct((R * N, D), _f32),
  )(h, Wrel)


def _tc_comb_rgcn(h, P0, P1, c0v, c1v, Wroot, b):
  RB = 1000
  NJ = N // RB

  def body(h_ref, p0, p1, c0, c1, w_ref, b_ref, o_ref):
    a = jnp.maximum(h_ref[...], 0.0)
    out = (jnp.dot(a, w_ref[...], preferred_element_type=_f32) + b_ref[...])
    q = p0[...] + p1[...]
    cnt = c0[...][:, :, 0] + c1[...][:, :, 0]
    for r in range(R):
      msum = jnp.concatenate([q[dc * R + r] for dc in range(4)], axis=1)
      out = out + msum / jnp.maximum(cnt[r], 1.0)[:, None]
    o_ref[...] = jnp.maximum(out, 0.0)

  return pl.pallas_call(
      body, grid=(NJ,),
      in_specs=[pl.BlockSpec((RB, D), lambda j: (j, 0)),
                pl.BlockSpec((16, RB, 32), lambda j: (0, j, 0)),
                pl.BlockSpec((16, RB, 32), lambda j: (0, j, 0)),
                pl.BlockSpec((R, RB, 16), lambda j: (0, j, 0)),
                pl.BlockSpec((R, RB, 16), lambda j: (0, j, 0)),
                pl.BlockSpec((D, D), lambda j: (0, 0)),
                pl.BlockSpec((1, D), lambda j: (0, 0))],
      out_specs=pl.BlockSpec((RB, D), lambda j: (j, 0)),
      out_shape=jax.ShapeDtypeStruct((N, D), _f32),
  )(h, P0, P1, c0v, c1v, Wroot, b.reshape(1, D))


def _tc_comb_mf(h2, Q0, Q1, c0v, c1v, Wl, bl, Wr):
  RB = 1000
  NJ = N // RB
  NK = MAXDEG + 1

  def body(h_ref, q0, q1, c0, c1, wl_ref, bl_ref, wr_ref, o_ref):
    h = h_ref[...]
    agg = q0[...] + q1[...]
    cnt = c0[...][:, :, 0] + c1[...][:, :, 0]
    deg = jnp.clip(cnt[0] + cnt[1] + cnt[2] + cnt[3], 0.0, float(MAXDEG))
    out = jnp.zeros((RB, D), _f32)
    for k in range(NK):
      t = (jnp.dot(agg, wl_ref[k], preferred_element_type=_f32)
           + jnp.dot(h, wr_ref[k], preferred_element_type=_f32)
           + bl_ref[k])
      out = jnp.where((deg == float(k))[:, None], t, out)
    o_ref[...] = out

  return pl.pallas_call(
      body, grid=(NJ,),
      in_specs=[pl.BlockSpec((RB, D), lambda j: (j, 0)),
                pl.BlockSpec((RB, D), lambda j: (j, 0)),
                pl.BlockSpec((RB, D), lambda j: (j, 0)),
                pl.BlockSpec((R, RB, 16), lambda j: (0, j, 0)),
                pl.BlockSpec((R, RB, 16), lambda j: (0, j, 0)),
                pl.BlockSpec((NK, D, D), lambda j: (0, 0, 0)),
                pl.BlockSpec((NK, 1, D), lambda j: (0, 0, 0)),
                pl.BlockSpec((NK, D, D), lambda j: (0, 0, 0))],
      out_specs=pl.BlockSpec((RB, D), lambda j: (j, 0)),
      out_shape=jax.ShapeDtypeStruct((N, D), _f32),
  )(h2, Q0, Q1, c0v, c1v, Wl, bl.reshape(NK, 1, D), Wr)


def _tc_pool_head(h, batch3, W_h1, b_h1, W_h2, b_h2):
  RB = 1000
  NJ = N // RB

  def body(h_ref, b_ref, w1, b1, w2, b2, o_ref, acc):
    j = pl.program_id(0)

    @pl.when(j == 0)
    def _():
      acc[...] = jnp.zeros((G, D), _f32)

    bi = b_ref[0]
    gi = lax.broadcasted_iota(jnp.int32, (G, RB), 0)
    oh = (bi == gi).astype(_f32)
    acc[...] += jnp.dot(oh, h_ref[...], preferred_element_type=_f32)

    @pl.when(j == NJ - 1)
    def _():
      hid = jnp.maximum(
          jnp.dot(acc[...], w1[...], preferred_element_type=_f32) + b1[...],
          0.0)
      o_ref[...] = jnp.dot(hid, w2[...], preferred_element_type=_f32) + b2[...]

  return pl.pallas_call(
      body, grid=(NJ,),
      in_specs=[pl.BlockSpec((RB, D), lambda j: (j, 0)),
                pl.BlockSpec((1, 1, RB), lambda j: (j, 0, 0)),
                pl.BlockSpec((D, D), lambda j: (0, 0)),
                pl.BlockSpec((1, D), lambda j: (0, 0)),
                pl.BlockSpec((D, D), lambda j: (0, 0)),
                pl.BlockSpec((1, D), lambda j: (0, 0))],
      out_specs=pl.BlockSpec((G, D), lambda j: (0, 0)),
      out_shape=jax.ShapeDtypeStruct((G, D), _f32),
      scratch_shapes=[pltpu.VMEM((G, D), _f32)],
  )(h, batch3, W_h1, b_h1.reshape(1, D), W_h2, b_h2.reshape(1, D))


def kernel(x, edge_index, edge_attr, batch_idx, W_in, b_in, W_rel, W_root,
           b_rgcn, W_l, b_l, W_r, W_h1, b_h1, W_h2, b_h2):
  src, dst, et = edge_index[0], edge_index[1], edge_attr
  padn = EPAD - E
  src_p = jnp.concatenate([src, jnp.zeros((padn,), jnp.int32)])
  dst_p = jnp.concatenate([dst, jnp.full((padn,), PAD_DST, jnp.int32)])
  et_p = jnp.concatenate([et, jnp.zeros((padn,), jnp.int32)])
  srcR = src_p.reshape(NW, NCH, CH)
  dstR = dst_p.reshape(NW, NCH, CH)
  etR = et_p.reshape(NW, NCH, CH)
  srcM = src_p.reshape(NW, NCHM, CHM)
  dstM = dst_p.reshape(NW, NCHM, CHM)

  mesh = _mesh()
  hist = _k_hist(mesh)(etR, dstR)
  c0v = hist[0].reshape(R, RSP, 16)
  c1v = hist[1].reshape(R, RSP, 16)

  h = _tc_in(x, W_in, b_in)
  for blk in range(2):
    Hcat = _tc_tables(h, W_rel[blk])
    tbl = Hcat.reshape(R * N * 4, 32)
    P = _k_rgcn(mesh)(tbl, etR, srcR, dstR)
    P0 = P[0].reshape(16, RSP, 32)
    P1 = P[1].reshape(16, RSP, 32)
    h2 = _tc_comb_rgcn(h, P0, P1, c0v, c1v, W_root[blk], b_rgcn[blk])
    Q = _k_mf(mesh)(h2, srcM, dstM)
    h = _tc_comb_mf(h2, Q[0], Q[1], c0v, c1v, W_l[blk], b_l[blk], W_r[blk])

  return _tc_pool_head(h, batch_idx.reshape(10, 1, N // 10),
                       W_h1, b_h1, W_h2, b_h2)


# speedup vs baseline: 6.4116x; 1.0298x over previous
import jax
import jax.numpy as jnp
from jax import lax
from jax.experimental import pallas as pl
from jax.experimental.pallas import tpu as pltpu
from jax.experimental.pallas import tpu_sc as plsc

N = 10000
E = 320000
D = 128
R = 4
G = 64
MAXDEG = 10

NC, NS = 2, 16
NW = NC * NS
CH = 128
NCH = 79
CHM = 32
NCHM = 316
NBUF = 4
EPT = NCH * CH
EPAD = NW * EPT
RSP = 10240
HB = R * RSP
PAD_DST = N

_f32 = jnp.float32


def _mesh():
  return plsc.VectorSubcoreMesh(
      core_axis_name="c", subcore_axis_name="s",
      num_cores=NC, num_subcores=NS)


_SC_PARAMS = pltpu.CompilerParams(use_tc_tiling_on_sc=False)


def _fill_rows(ref, nrows, ncol16, value):
  v = jnp.full((16,), value, _f32)

  @pl.loop(0, nrows)
  def _(i):
    for k in range(ncol16):
      ref[i, pl.ds(k * 16, 16)] = v


def _hist_body(etR, srcR, dstR, out, idxgR, idxsR, et2, src2, dst2, ones_v,
               z_v, dmp_v, hist_sh, hsem):
  cid = lax.axis_index("c")
  sid = lax.axis_index("s")
  w = cid * NS + sid
  myrow = sid * (HB // NS)
  _fill_rows(ones_v, CH, 1, 1.0)
  _fill_rows(z_v, CH, 1, 0.0)

  @pl.loop(0, HB // NS // CH)
  def _(j):
    pltpu.sync_copy(z_v, hist_sh.at[pl.ds(myrow + j * CH, CH)])

  pltpu.sync_copy(etR.at[w], et2)
  pltpu.sync_copy(srcR.at[w], src2)
  pltpu.sync_copy(dstR.at[w], dst2)

  @pl.loop(0, NCH)
  def _(j):
    for k in range(CH // 16):
      sl = pl.ds(k * 16, 16)
      et16 = et2[j, sl]
      src2[j, sl] = et16 * (4 * N) + src2[j, sl] * 4
      dst2[j, sl] = et16 * RSP + dst2[j, sl]

  pltpu.sync_copy(src2, idxgR.at[w])
  pltpu.sync_copy(dst2, idxsR.at[w])
  plsc.subcore_barrier()

  @pl.loop(0, NCH)
  def _(j):
    pltpu.async_copy(ones_v, hist_sh.at[dst2.at[j]], hsem, add=True)

    @pl.when(j >= 8)
    def _():
      pltpu.make_async_copy(ones_v, hist_sh.at[dst2.at[j - 8]], hsem).wait()

  @pl.loop(NCH - 8, NCH)
  def _(j):
    pltpu.make_async_copy(ones_v, hist_sh.at[dst2.at[j]], hsem).wait()

  plsc.subcore_barrier()

  @pl.loop(0, HB // NS // CH)
  def _(j):
    pltpu.sync_copy(hist_sh.at[pl.ds(myrow + j * CH, CH)], dmp_v)
    pltpu.sync_copy(dmp_v, out.at[cid, pl.ds(myrow + j * CH, CH)])


def _k_hist(mesh):
  return pl.kernel(
      _hist_body,
      out_type=(jax.ShapeDtypeStruct((NC, HB, 16), _f32),
                jax.ShapeDtypeStruct((NW, NCH, CH), jnp.int32),
                jax.ShapeDtypeStruct((NW, NCH, CH), jnp.int32)),
      mesh=mesh,
      compiler_params=_SC_PARAMS,
      scratch_types=[
          pltpu.VMEM((NCH, CH), jnp.int32),
          pltpu.VMEM((NCH, CH), jnp.int32),
          pltpu.VMEM((NCH, CH), jnp.int32),
          pltpu.VMEM((CH, 16), _f32),
          pltpu.VMEM((CH, 16), _f32),
          pltpu.VMEM((CH, 16), _f32),
          pltpu.VMEM_SHARED((HB, 16), _f32),
          pltpu.SemaphoreType.DMA,
      ])


def _gs_pipeline(tbl, idxg_row, idxs_row, rows, acc_sh, gsem, ssem, nch):
  def gather(j):
    b = lax.rem(j, NBUF)
    pltpu.async_copy(tbl.at[idxg_row(j)], rows.at[b], gsem.at[b])

  def wait_gather(j):
    b = lax.rem(j, NBUF)
    pltpu.make_async_copy(tbl.at[idxg_row(j)], rows.at[b], gsem.at[b]).wait()

  def scatter(j):
    b = lax.rem(j, NBUF)
    pltpu.async_copy(rows.at[b], acc_sh.at[idxs_row(j)], ssem.at[b], add=True)

  def wait_scatter(j):
    b = lax.rem(j, NBUF)
    pltpu.make_async_copy(
        rows.at[b], acc_sh.at[idxs_row(j)], ssem.at[b]).wait()

  gather(0)
  gather(1)

  @pl.loop(0, nch)
  def _(j):
    @pl.when(j >= 2)
    def _():
      wait_scatter(j - 2)

    @pl.when(j + 2 < nch)
    def _():
      gather(j + 2)

    wait_gather(j)
    scatter(j)

  wait_scatter(nch - 2)
  wait_scatter(nch - 1)


def _rgcn_body(tbl, idxgR, idxsR, out, idxg, idxs, rows, acc_sh, gsem, ssem):
  cid = lax.axis_index("c")
  sid = lax.axis_index("s")
  w = cid * NS + sid
  myrow = sid * (HB // NS)
  pltpu.sync_copy(idxgR.at[w], idxg)
  pltpu.sync_copy(idxsR.at[w], idxs)

  for dc in range(4):
    if dc > 0:
      @pl.loop(0, NCH)
      def _(j):
        for k in range(CH // 16):
          sl = pl.ds(k * 16, 16)
          idxg[j, sl] = idxg[j, sl] + 1

    _fill_rows(rows.at[0], CH, 2, 0.0)

    @pl.loop(0, HB // NS // CH)
    def _(j):
      pltpu.sync_copy(rows.at[0], acc_sh.at[pl.ds(myrow + j * CH, CH)])

    plsc.subcore_barrier()
    _gs_pipeline(tbl, lambda j: idxg.at[j], lambda j: idxs.at[j],
                 rows, acc_sh, gsem, ssem, NCH)
    plsc.subcore_barrier()

    @pl.loop(0, HB // NS // CH)
    def _(j):
      pltpu.sync_copy(acc_sh.at[pl.ds(myrow + j * CH, CH)], rows.at[0])
      pltpu.sync_copy(rows.at[0], out.at[cid, dc, pl.ds(myrow + j * CH, CH)])


def _k_rgcn(mesh):
  return pl.kernel(
      _rgcn_body,
      out_type=jax.ShapeDtypeStruct((NC, 4, HB, 32), _f32),
      mesh=mesh,
      compiler_params=_SC_PARAMS,
      scratch_types=[
          pltpu.VMEM((NCH, CH), jnp.int32),
          pltpu.VMEM((NCH, CH), jnp.int32),
          pltpu.VMEM((NBUF, CH, 32), _f32),
          pltpu.VMEM_SHARED((HB, 32), _f32),
          pltpu.SemaphoreType.DMA((NBUF,)),
          pltpu.SemaphoreType.DMA((NBUF,)),
      ])


def _mf_body(tbl, srcR, dstR, out, src2, dst2, rows, acc_sh, gsem, ssem):
  cid = lax.axis_index("c")
  sid = lax.axis_index("s")
  w = cid * NS + sid
  myrow = sid * (RSP // NS)
  pltpu.sync_copy(srcR.at[w], src2)
  pltpu.sync_copy(dstR.at[w], dst2)
  _fill_rows(rows.at[0], CHM, D // 16, 0.0)

  @pl.loop(0, RSP // NS // CHM)
  def _(j):
    pltpu.sync_copy(rows.at[0], acc_sh.at[pl.ds(myrow + j * CHM, CHM)])

  plsc.subcore_barrier()
  _gs_pipeline(tbl, lambda j: src2.at[j], lambda j: dst2.at[j],
               rows, acc_sh, gsem, ssem, NCHM)
  plsc.subcore_barrier()

  @pl.loop(0, RSP // NS // CHM)
  def _(j):
    pltpu.sync_copy(acc_sh.at[pl.ds(myrow + j * CHM, CHM)], rows.at[0])
    pltpu.sync_copy(rows.at[0], out.at[cid, pl.ds(myrow + j * CHM, CHM)])


def _k_mf(mesh):
  return pl.kernel(
      _mf_body,
      out_type=jax.ShapeDtypeStruct((NC, RSP, D), _f32),
      mesh=mesh,
      compiler_params=_SC_PARAMS,
      scratch_types=[
          pltpu.VMEM((NCHM, CHM), jnp.int32),
          pltpu.VMEM((NCHM, CHM), jnp.int32),
          pltpu.VMEM((NBUF, CHM, D), _f32),
          pltpu.VMEM_SHARED((RSP, D), _f32),
          pltpu.SemaphoreType.DMA((NBUF,)),
          pltpu.SemaphoreType.DMA((NBUF,)),
      ])


def _tc_in(x, W, b):
  RB = 1000

  def body(x_ref, w_ref, b_ref, o_ref):
    o_ref[...] = (jnp.dot(x_ref[...], w_ref[...],
                          preferred_element_type=_f32) + b_ref[...])

  return pl.pallas_call(
      body, grid=(N // RB,),
      in_specs=[pl.BlockSpec((RB, D), lambda j: (j, 0)),
                pl.BlockSpec((D, D), lambda j: (0, 0)),
                pl.BlockSpec((1, D), lambda j: (0, 0))],
      out_specs=pl.BlockSpec((RB, D), lambda j: (j, 0)),
      out_shape=jax.ShapeDtypeStruct((N, D), _f32),
  )(x, W, b.reshape(1, D))


def _tc_tables(h, Wrel):
  RB = 1000

  def body(h_ref, w_ref, o_ref):
    a = jnp.maximum(h_ref[...], 0.0)
    o_ref[...] = jnp.dot(a, w_ref[0], preferred_element_type=_f32)

  return pl.pallas_call(
      body, grid=(R, N // RB),
      in_specs=[pl.BlockSpec((RB, D), lambda r, j: (j, 0)),
                pl.BlockSpec((1, D, D), lambda r, j: (r, 0, 0))],
      out_specs=pl.BlockSpec((RB, D), lambda r, j: (r * (N // RB) + j, 0)),
      out_shape=jax.ShapeDtypeStruct((R * N, D), _f32),
  )(h, Wrel)


def _tc_comb_rgcn(h, P0, P1, c0v, c1v, Wroot, b):
  RB = 1000
  NJ = N // RB

  def body(h_ref, p0, p1, c0, c1, w_ref, b_ref, o_ref):
    a = jnp.maximum(h_ref[...], 0.0)
    out = (jnp.dot(a, w_ref[...], preferred_element_type=_f32) + b_ref[...])
    q = p0[...] + p1[...]
    cnt = c0[...][:, :, 0] + c1[...][:, :, 0]
    for r in range(R):
      msum = jnp.concatenate([q[dc * R + r] for dc in range(4)], axis=1)
      out = out + msum / jnp.maximum(cnt[r], 1.0)[:, None]
    o_ref[...] = jnp.maximum(out, 0.0)

  return pl.pallas_call(
      body, grid=(NJ,),
      in_specs=[pl.BlockSpec((RB, D), lambda j: (j, 0)),
                pl.BlockSpec((16, RB, 32), lambda j: (0, j, 0)),
                pl.BlockSpec((16, RB, 32), lambda j: (0, j, 0)),
                pl.BlockSpec((R, RB, 16), lambda j: (0, j, 0)),
                pl.BlockSpec((R, RB, 16), lambda j: (0, j, 0)),
                pl.BlockSpec((D, D), lambda j: (0, 0)),
                pl.BlockSpec((1, D), lambda j: (0, 0))],
      out_specs=pl.BlockSpec((RB, D), lambda j: (j, 0)),
      out_shape=jax.ShapeDtypeStruct((N, D), _f32),
  )(h, P0, P1, c0v, c1v, Wroot, b.reshape(1, D))


def _tc_comb_mf(h2, Q0, Q1, c0v, c1v, Wl, bl, Wr):
  RB = 1000
  NJ = N // RB
  NK = MAXDEG + 1

  def body(h_ref, q0, q1, c0, c1, wl_ref, bl_ref, wr_ref, o_ref):
    h = h_ref[...]
    agg = q0[...] + q1[...]
    cnt = c0[...][:, :, 0] + c1[...][:, :, 0]
    deg = jnp.clip(cnt[0] + cnt[1] + cnt[2] + cnt[3], 0.0, float(MAXDEG))
    out = jnp.zeros((RB, D), _f32)
    for k in range(NK):
      t = (jnp.dot(agg, wl_ref[k], preferred_element_type=_f32)
           + jnp.dot(h, wr_ref[k], preferred_element_type=_f32)
           + bl_ref[k])
      out = jnp.where((deg == float(k))[:, None], t, out)
    o_ref[...] = out

  return pl.pallas_call(
      body, grid=(NJ,),
      in_specs=[pl.BlockSpec((RB, D), lambda j: (j, 0)),
                pl.BlockSpec((RB, D), lambda j: (j, 0)),
                pl.BlockSpec((RB, D), lambda j: (j, 0)),
                pl.BlockSpec((R, RB, 16), lambda j: (0, j, 0)),
                pl.BlockSpec((R, RB, 16), lambda j: (0, j, 0)),
                pl.BlockSpec((NK, D, D), lambda j: (0, 0, 0)),
                pl.BlockSpec((NK, 1, D), lambda j: (0, 0, 0)),
                pl.BlockSpec((NK, D, D), lambda j: (0, 0, 0))],
      out_specs=pl.BlockSpec((RB, D), lambda j: (j, 0)),
      out_shape=jax.ShapeDtypeStruct((N, D), _f32),
  )(h2, Q0, Q1, c0v, c1v, Wl, bl.reshape(NK, 1, D), Wr)


def _tc_pool_head(h, batch3, W_h1, b_h1, W_h2, b_h2):
  RB = 1000
  NJ = N // RB

  def body(h_ref, b_ref, w1, b1, w2, b2, o_ref, acc):
    j = pl.program_id(0)

    @pl.when(j == 0)
    def _():
      acc[...] = jnp.zeros((G, D), _f32)

    bi = b_ref[0]
    gi = lax.broadcasted_iota(jnp.int32, (G, RB), 0)
    oh = (bi == gi).astype(_f32)
    acc[...] += jnp.dot(oh, h_ref[...], preferred_element_type=_f32)

    @pl.when(j == NJ - 1)
    def _():
      hid = jnp.maximum(
          jnp.dot(acc[...], w1[...], preferred_element_type=_f32) + b1[...],
          0.0)
      o_ref[...] = jnp.dot(hid, w2[...], preferred_element_type=_f32) + b2[...]

  return pl.pallas_call(
      body, grid=(NJ,),
      in_specs=[pl.BlockSpec((RB, D), lambda j: (j, 0)),
                pl.BlockSpec((1, 1, RB), lambda j: (j, 0, 0)),
                pl.BlockSpec((D, D), lambda j: (0, 0)),
                pl.BlockSpec((1, D), lambda j: (0, 0)),
                pl.BlockSpec((D, D), lambda j: (0, 0)),
                pl.BlockSpec((1, D), lambda j: (0, 0))],
      out_specs=pl.BlockSpec((G, D), lambda j: (0, 0)),
      out_shape=jax.ShapeDtypeStruct((G, D), _f32),
      scratch_shapes=[pltpu.VMEM((G, D), _f32)],
  )(h, batch3, W_h1, b_h1.reshape(1, D), W_h2, b_h2.reshape(1, D))


def kernel(x, edge_index, edge_attr, batch_idx, W_in, b_in, W_rel, W_root,
           b_rgcn, W_l, b_l, W_r, W_h1, b_h1, W_h2, b_h2):
  src, dst, et = edge_index[0], edge_index[1], edge_attr
  padn = EPAD - E
  src_p = jnp.concatenate([src, jnp.zeros((padn,), jnp.int32)])
  dst_p = jnp.concatenate([dst, jnp.full((padn,), PAD_DST, jnp.int32)])
  et_p = jnp.concatenate([et, jnp.zeros((padn,), jnp.int32)])
  srcR = src_p.reshape(NW, NCH, CH)
  dstR = dst_p.reshape(NW, NCH, CH)
  etR = et_p.reshape(NW, NCH, CH)
  srcM = src_p.reshape(NW, NCHM, CHM)
  dstM = dst_p.reshape(NW, NCHM, CHM)

  mesh = _mesh()
  hist, idxgR, idxsR = _k_hist(mesh)(etR, srcR, dstR)
  c0v = hist[0].reshape(R, RSP, 16)
  c1v = hist[1].reshape(R, RSP, 16)

  h = _tc_in(x, W_in, b_in)
  for blk in range(2):
    Hcat = _tc_tables(h, W_rel[blk])
    tbl = Hcat.reshape(R * N * 4, 32)
    P = _k_rgcn(mesh)(tbl, idxgR, idxsR)
    P0 = P[0].reshape(16, RSP, 32)
    P1 = P[1].reshape(16, RSP, 32)
    h2 = _tc_comb_rgcn(h, P0, P1, c0v, c1v, W_root[blk], b_rgcn[blk])
    Q = _k_mf(mesh)(h2, srcM, dstM)
    h = _tc_comb_mf(h2, Q[0], Q[1], c0v, c1v, W_l[blk], b_l[blk], W_r[blk])

  return _tc_pool_head(h, batch_idx.reshape(10, 1, N // 10),
                       W_h1, b_h1, W_h2, b_h2)
